# baseline (speedup 1.0000x reference)
import jax
import jax.numpy as jnp
from jax.experimental import pallas as pl

_VPAD = 128
_EMB = 128
_HID2 = 512
_OUT = 256


def _prep_kernel(tables_ref, w1at_ref, w1bt_ref, ab_ref):
    a = jnp.dot(tables_ref[0:_VPAD, :], w1at_ref[...],
                preferred_element_type=jnp.float32)
    b = jnp.dot(tables_ref[_VPAD : 2 * _VPAD, :], w1bt_ref[...],
                preferred_element_type=jnp.float32)
    ab_ref[0:_VPAD, :] = a.astype(jnp.bfloat16)
    ab_ref[_VPAD : 2 * _VPAD, :] = b.astype(jnp.bfloat16)


def _main_kernel(idx_ref, ab_ref, b1_ref, w2t_ref, b2_ref, out_ref):
    bn = idx_ref.shape[0]
    idx2 = idx_ref[:, 2:3]
    idx3 = idx_ref[:, 3:4] + _VPAD
    iota = jax.lax.broadcasted_iota(jnp.int32, (bn, 2 * _VPAD), 1)
    oh = ((iota == idx2) | (iota == idx3)).astype(jnp.bfloat16)
    h = jnp.dot(oh, ab_ref[...], preferred_element_type=jnp.float32)
    h = jnp.maximum(h + b1_ref[...], 0.0).astype(jnp.bfloat16)
    out_ref[...] = (
        jnp.dot(h, w2t_ref[...], preferred_element_type=jnp.float32) + b2_ref[...]
    )


def kernel(input_tensor, meter_table, unit_table, W1, b1, W2, b2):
    n = input_tensor.shape[0]
    bn = 2048
    meter_pad = jnp.pad(meter_table, ((0, _VPAD - meter_table.shape[0]), (0, 0)))
    unit_pad = jnp.pad(unit_table, ((0, _VPAD - unit_table.shape[0]), (0, 0)))
    tables = jnp.concatenate([meter_pad, unit_pad], axis=0)
    w1at = W1[:, :_EMB].T
    w1bt = W1[:, _EMB:].T
    w2t = W2.T

    ab = pl.pallas_call(
        _prep_kernel,
        out_shape=jax.ShapeDtypeStruct((2 * _VPAD, _HID2), jnp.bfloat16),
    )(tables, w1at, w1bt)

    out = pl.pallas_call(
        _main_kernel,
        grid=(n // bn,),
        in_specs=[
            pl.BlockSpec((bn, 4), lambda i: (i, 0)),
            pl.BlockSpec((2 * _VPAD, _HID2), lambda i: (0, 0)),
            pl.BlockSpec((1, _HID2), lambda i: (0, 0)),
            pl.BlockSpec((_HID2, _OUT), lambda i: (0, 0)),
            pl.BlockSpec((1, _OUT), lambda i: (0, 0)),
        ],
        out_specs=pl.BlockSpec((bn, _OUT), lambda i: (i, 0)),
        out_shape=jax.ShapeDtypeStruct((n, _OUT), jnp.float32),
    )(input_tensor, ab, b1.reshape(1, _HID2), w2t.astype(jnp.bfloat16),
      b2.reshape(1, _OUT))
    return out


# speedup vs baseline: 1.3383x; 1.3383x over previous
import jax
import jax.numpy as jnp
from jax.experimental import pallas as pl
from jax.experimental.pallas import tpu as pltpu

_VPAD = 128
_EMB = 128
_HID2 = 512
_OUT = 256

_NT = (((1,), (1,)), ((), ()))


def _fused_kernel(idx_ref, meter_ref, unit_ref, w1_ref, b1_ref, w2_ref, b2_ref,
                  out_ref, ab_ref, w2t_ref):
    @pl.when(pl.program_id(0) == 0)
    def _prep():
        a = jax.lax.dot_general(meter_ref[...], w1_ref[:, :_EMB], _NT,
                                preferred_element_type=jnp.float32)
        b = jax.lax.dot_general(unit_ref[...], w1_ref[:, _EMB:], _NT,
                                preferred_element_type=jnp.float32)
        npad = _VPAD - a.shape[0]
        ab_ref[...] = jnp.concatenate(
            [jnp.pad(a, ((0, npad), (0, 0))), jnp.pad(b, ((0, npad), (0, 0)))],
            axis=0).astype(jnp.bfloat16)
        w2t_ref[...] = w2_ref[...].T.astype(jnp.bfloat16)

    bn = idx_ref.shape[0]
    idx2 = idx_ref[:, 2:3]
    idx3 = idx_ref[:, 3:4] + _VPAD
    iota = jax.lax.broadcasted_iota(jnp.int32, (bn, 2 * _VPAD), 1)
    oh = ((iota == idx2) | (iota == idx3)).astype(jnp.bfloat16)
    h = jnp.dot(oh, ab_ref[...], preferred_element_type=jnp.float32)
    h = jnp.maximum(h + b1_ref[...], 0.0).astype(jnp.bfloat16)
    out_ref[...] = (
        jnp.dot(h, w2t_ref[...], preferred_element_type=jnp.float32) + b2_ref[...]
    )


def kernel(input_tensor, meter_table, unit_table, W1, b1, W2, b2):
    n = input_tensor.shape[0]
    bn = 2048
    v_meter = meter_table.shape[0]
    v_unit = unit_table.shape[0]

    out = pl.pallas_call(
        _fused_kernel,
        grid=(n // bn,),
        in_specs=[
            pl.BlockSpec((bn, 4), lambda i: (i, 0)),
            pl.BlockSpec((v_meter, _EMB), lambda i: (0, 0)),
            pl.BlockSpec((v_unit, _EMB), lambda i: (0, 0)),
            pl.BlockSpec((_HID2, 2 * _EMB), lambda i: (0, 0)),
            pl.BlockSpec((1, _HID2), lambda i: (0, 0)),
            pl.BlockSpec((_OUT, _HID2), lambda i: (0, 0)),
            pl.BlockSpec((1, _OUT), lambda i: (0, 0)),
        ],
        out_specs=pl.BlockSpec((bn, _OUT), lambda i: (i, 0)),
        out_shape=jax.ShapeDtypeStruct((n, _OUT), jnp.float32),
        scratch_shapes=[
            pltpu.VMEM((2 * _VPAD, _HID2), jnp.bfloat16),
            pltpu.VMEM((_HID2, _OUT), jnp.bfloat16),
        ],
    )(input_tensor, meter_table, unit_table, W1,
      b1.reshape(1, _HID2), W2, b2.reshape(1, _OUT))
    return out


# speedup vs baseline: 1.4123x; 1.0553x over previous
import jax
import jax.numpy as jnp
from jax.experimental import pallas as pl
from jax.experimental.pallas import tpu as pltpu

_VPAD = 128
_EMB = 128
_HID2 = 512
_OUT = 256

_NT = (((1,), (1,)), ((), ()))


def _fused_kernel(idx_ref, meter_ref, unit_ref, w1_ref, b1_ref, w2_ref, b2_ref,
                  out_ref, ab_ref, w2t_ref, b1c_ref):
    @pl.when(pl.program_id(0) == 0)
    def _prep():
        a = jax.lax.dot_general(meter_ref[...], w1_ref[:, :_EMB], _NT,
                                preferred_element_type=jnp.float32)
        b = jax.lax.dot_general(unit_ref[...], w1_ref[:, _EMB:], _NT,
                                preferred_element_type=jnp.float32)
        npad = _VPAD - a.shape[0]
        ab_ref[...] = jnp.concatenate(
            [jnp.pad(a, ((0, npad), (0, 0))), jnp.pad(b, ((0, npad), (0, 0)))],
            axis=0).astype(jnp.bfloat16)
        w2t_ref[...] = w2_ref[...].T.astype(jnp.bfloat16)
        b1c_ref[...] = b1_ref[...].astype(jnp.bfloat16)

    bn = idx_ref.shape[0]
    idx2 = idx_ref[:, 2:3]
    idx3 = idx_ref[:, 3:4] + _VPAD
    iota = jax.lax.broadcasted_iota(jnp.int32, (bn, 2 * _VPAD), 1)
    oh = ((iota == idx2) | (iota == idx3)).astype(jnp.bfloat16)
    h = jnp.dot(oh, ab_ref[...], preferred_element_type=jnp.float32)
    h = jnp.maximum(h.astype(jnp.bfloat16) + b1c_ref[...], jnp.bfloat16(0.0))
    out_ref[...] = (
        jnp.dot(h, w2t_ref[...], preferred_element_type=jnp.float32) + b2_ref[...]
    )


def kernel(input_tensor, meter_table, unit_table, W1, b1, W2, b2):
    n = input_tensor.shape[0]
    bn = 4096
    v_meter = meter_table.shape[0]
    v_unit = unit_table.shape[0]

    out = pl.pallas_call(
        _fused_kernel,
        grid=(n // bn,),
        in_specs=[
            pl.BlockSpec((bn, 4), lambda i: (i, 0)),
            pl.BlockSpec((v_meter, _EMB), lambda i: (0, 0)),
            pl.BlockSpec((v_unit, _EMB), lambda i: (0, 0)),
            pl.BlockSpec((_HID2, 2 * _EMB), lambda i: (0, 0)),
            pl.BlockSpec((1, _HID2), lambda i: (0, 0)),
            pl.BlockSpec((_OUT, _HID2), lambda i: (0, 0)),
            pl.BlockSpec((1, _OUT), lambda i: (0, 0)),
        ],
        out_specs=pl.BlockSpec((bn, _OUT), lambda i: (i, 0)),
        out_shape=jax.ShapeDtypeStruct((n, _OUT), jnp.float32),
        scratch_shapes=[
            pltpu.VMEM((2 * _VPAD, _HID2), jnp.bfloat16),
            pltpu.VMEM((_HID2, _OUT), jnp.bfloat16),
            pltpu.VMEM((1, _HID2), jnp.bfloat16),
        ],
    )(input_tensor, meter_table, unit_table, W1,
      b1.reshape(1, _HID2), W2, b2.reshape(1, _OUT))
    return out
